# baseline (speedup 1.0000x reference)
import functools

import jax
import jax.numpy as jnp
from jax import lax
from jax.experimental import pallas as pl
from jax.experimental.pallas import tpu as pltpu
from jax.experimental.pallas import tpu_sc as plsc

N_PAD = 10240
HALF_STRIDE = 11264
AGG_ROWS = 10368
ROWS_PER_TILE = 648
E_PAD = 102400
EDGES_PER_TILE = 6400
BLK = 128
NBLK = EDGES_PER_TILE // BLK
R = 1024
NRB = N_PAD // R

_SC_MESH = plsc.VectorSubcoreMesh(core_axis_name="c", subcore_axis_name="s")


def _zero_tile_region(gbuf, agg, base):
    zero16 = jnp.zeros((16,), jnp.float32)

    def _zrow(i, carry):
        for c in range(8):
            gbuf[i, pl.ds(c * 16, 16)] = zero16
        return carry
    lax.fori_loop(0, BLK, _zrow, 0)
    for k in range(ROWS_PER_TILE // BLK):
        pltpu.sync_copy(gbuf.at[pl.ds(0, BLK)],
                        agg.at[pl.ds(base + k * BLK, BLK)])
    rem = ROWS_PER_TILE % BLK
    if rem:
        pltpu.sync_copy(gbuf.at[pl.ds(0, rem)],
                        agg.at[pl.ds(base + ROWS_PER_TILE - rem, rem)])


NBUF = 2


def _segsum_body(v_hbm, src_hbm, dst_hbm, out_hbm,
                 gbuf, gidx_all, didx_all, agg, gsem, ssem):
    cid = lax.axis_index("c")
    sid = lax.axis_index("s")
    base = sid * ROWS_PER_TILE

    pltpu.sync_copy(src_hbm.at[cid].at[sid], gidx_all)
    pltpu.sync_copy(dst_hbm.at[sid], didx_all)
    _zero_tile_region(gbuf, agg, base)
    plsc.subcore_barrier()

    def _slot(b):
        return gbuf.at[pl.ds(b * BLK, BLK)]

    def _gather(j, b):
        pltpu.async_copy(v_hbm.at[gidx_all.at[j]], _slot(b), gsem)

    def _gather_wait(j, b):
        pltpu.make_async_copy(v_hbm.at[gidx_all.at[j]], _slot(b), gsem).wait()

    def _scatter(j, b):
        pltpu.async_copy(_slot(b), agg.at[didx_all.at[j]], ssem, add=True)

    def _scatter_wait(j, b):
        pltpu.make_async_copy(_slot(b), agg.at[didx_all.at[j]], ssem).wait()

    for b in range(NBUF):
        _gather(b, b)

    def _round(it, carry):
        j0 = it * NBUF
        for b in range(NBUF):
            _gather_wait(j0 + b, b)
            _scatter(j0 + b, b)
        for b in range(NBUF):
            _scatter_wait(j0 + b, b)
            _gather(j0 + b + NBUF, b)
        return carry

    lax.fori_loop(0, NBLK // NBUF - 1, _round, 0)

    j0 = NBLK - NBUF
    for b in range(NBUF):
        _gather_wait(j0 + b, b)
        _scatter(j0 + b, b)
    for b in range(NBUF):
        _scatter_wait(j0 + b, b)

    plsc.subcore_barrier()

    pltpu.sync_copy(agg.at[pl.ds(base, ROWS_PER_TILE)],
                    out_hbm.at[pl.ds(cid * HALF_STRIDE + base, ROWS_PER_TILE)])


_segsum = pl.kernel(
    _segsum_body,
    mesh=_SC_MESH,
    out_type=jax.ShapeDtypeStruct((2 * HALF_STRIDE, 128), jnp.float32),
    scratch_types=[
        pltpu.VMEM((NBUF * BLK, 128), jnp.float32),
        pltpu.VMEM((NBLK, BLK), jnp.int32),
        pltpu.VMEM((NBLK, BLK), jnp.int32),
        pltpu.VMEM_SHARED((AGG_ROWS, 128), jnp.float32),
        pltpu.SemaphoreType.DMA,
        pltpu.SemaphoreType.DMA,
    ],
)


def _count_body(dst_hbm, out_hbm, gbuf, didx_all, agg, ssem):
    cid = lax.axis_index("c")
    sid = lax.axis_index("s")
    base = sid * ROWS_PER_TILE

    _zero_tile_region(gbuf, agg, base)
    pltpu.sync_copy(dst_hbm.at[sid], didx_all)

    one16 = jnp.full((16,), 1.0, jnp.float32)

    def _orow(i, carry):
        for c in range(8):
            gbuf[i, pl.ds(c * 16, 16)] = one16
        return carry
    lax.fori_loop(0, BLK, _orow, 0)
    plsc.subcore_barrier()

    ones = gbuf

    def _round(it, carry):
        j0 = it * NBUF
        for b in range(NBUF):
            pltpu.async_copy(ones, agg.at[didx_all.at[j0 + b]], ssem,
                             add=True)
        for b in range(NBUF):
            pltpu.make_async_copy(ones, agg.at[didx_all.at[j0 + b]],
                                  ssem).wait()
        return carry

    lax.fori_loop(0, NBLK // NBUF, _round, 0)
    plsc.subcore_barrier()

    pltpu.sync_copy(agg.at[pl.ds(base, ROWS_PER_TILE)],
                    out_hbm.at[pl.ds(cid * HALF_STRIDE + base, ROWS_PER_TILE)])


_count = pl.kernel(
    _count_body,
    mesh=_SC_MESH,
    out_type=jax.ShapeDtypeStruct((2 * HALF_STRIDE, 128), jnp.float32),
    scratch_types=[
        pltpu.VMEM((BLK, 128), jnp.float32),
        pltpu.VMEM((NBLK, BLK), jnp.int32),
        pltpu.VMEM_SHARED((AGG_ROWS, 128), jnp.float32),
        pltpu.SemaphoreType.DMA,
    ],
)


_DN = (((1,), (1,)), ((), ()))


def _mm0_body(x_ref, w_ref, o_ref):
    o_ref[...] = lax.dot_general(x_ref[...], w_ref[...], _DN,
                                 preferred_element_type=jnp.float32)


def _mm0(x, wl0):
    return pl.pallas_call(
        _mm0_body,
        grid=(NRB, 2),
        in_specs=[pl.BlockSpec((R, 768), lambda i, j: (i, 0)),
                  pl.BlockSpec((128, 768), lambda i, j: (j, 0))],
        out_specs=pl.BlockSpec((R, 128), lambda i, j: (j * NRB + i, 0)),
        out_shape=jax.ShapeDtypeStruct((2 * N_PAD, 128), jnp.float32),
    )(x, wl0)


def _q0_body(x_ref, w_ref, b_ref, o_ref):
    o_ref[...] = b_ref[0] + lax.dot_general(
        x_ref[...], w_ref[...], _DN, preferred_element_type=jnp.float32)


def _q0(x, w, b):
    return pl.pallas_call(
        _q0_body,
        grid=(NRB, 2),
        in_specs=[
            pl.BlockSpec((R, 768), lambda i, j: (i, 0)),
            pl.BlockSpec((128, 768), lambda i, j: (j, 0)),
            pl.BlockSpec((1, 1, 128), lambda i, j: (j, 0, 0)),
        ],
        out_specs=pl.BlockSpec((R, 128), lambda i, j: (j * NRB + i, 0)),
        out_shape=jax.ShapeDtypeStruct((2 * N_PAD, 128), jnp.float32),
    )(x, w, b.reshape(2, 1, 128))


def _combine0_body(s_ref, c_ref, q_ref, o_ref):
    rd = 1.0 / jnp.maximum(c_ref[...][:, 0:1], 1.0)
    o_ref[...] = jnp.maximum(s_ref[...] * rd + q_ref[...], 0.0)


def _combine0(s, cnt, q):
    hs_blocks = HALF_STRIDE // R
    return pl.pallas_call(
        _combine0_body,
        grid=(NRB, 2),
        in_specs=[
            pl.BlockSpec((R, 128), lambda i, j: (j * hs_blocks + i, 0)),
            pl.BlockSpec((R, 128), lambda i, j: (i, 0)),
            pl.BlockSpec((R, 128), lambda i, j: (j * NRB + i, 0)),
        ],
        out_specs=pl.BlockSpec((R, 128), lambda i, j: (j * NRB + i, 0)),
        out_shape=jax.ShapeDtypeStruct((2 * N_PAD, 128), jnp.float32),
    )(s, cnt, q)


def _out_spec_shape(dout, out_sh):
    nj = dout // 128
    if out_sh:
        return (pl.BlockSpec((R, 128), lambda i, j: (j * NRB + i, 0)),
                jax.ShapeDtypeStruct((2 * N_PAD, 128), jnp.float32))
    return (pl.BlockSpec((R, 128), lambda i, j: (i, j)),
            jax.ShapeDtypeStruct((N_PAD, dout), jnp.float32))


def _layer_body(relu, sA, sB, c_ref, hA, hB, wl_ref, wr_ref, b_ref, o_ref):
    rd = 1.0 / jnp.maximum(c_ref[...][:, 0:1], 1.0)
    wl = wl_ref[...]
    wr = wr_ref[...]
    acc = lax.dot_general(sA[...] * rd, wl[:, :128], _DN,
                          preferred_element_type=jnp.float32)
    acc = acc + lax.dot_general(sB[...] * rd, wl[:, 128:], _DN,
                                preferred_element_type=jnp.float32)
    acc = acc + lax.dot_general(hA[...], wr[:, :128], _DN,
                                preferred_element_type=jnp.float32)
    acc = acc + lax.dot_general(hB[...], wr[:, 128:], _DN,
                                preferred_element_type=jnp.float32)
    acc = acc + b_ref[0]
    o_ref[...] = jnp.maximum(acc, 0.0) if relu else acc


def _layer(s, cnt, h, wl, wr, b, relu, out_sh):
    dout = wl.shape[0]
    nj = dout // 128
    hs_blocks = HALF_STRIDE // R
    out_spec, out_shape = _out_spec_shape(dout, out_sh)
    return pl.pallas_call(
        functools.partial(_layer_body, relu),
        grid=(NRB, nj),
        in_specs=[
            pl.BlockSpec((R, 128), lambda i, j: (i, 0)),
            pl.BlockSpec((R, 128), lambda i, j: (hs_blocks + i, 0)),
            pl.BlockSpec((R, 128), lambda i, j: (i, 0)),
            pl.BlockSpec((R, 128), lambda i, j: (i, 0)),
            pl.BlockSpec((R, 128), lambda i, j: (NRB + i, 0)),
            pl.BlockSpec((128, 256), lambda i, j: (j, 0)),
            pl.BlockSpec((128, 256), lambda i, j: (j, 0)),
            pl.BlockSpec((1, 1, 128), lambda i, j: (j, 0, 0)),
        ],
        out_specs=out_spec,
        out_shape=out_shape,
    )(s, s, cnt, h, h, wl, wr, b.reshape(nj, 1, 128))


def kernel(x, edge_index, params):
    x = x.astype(jnp.float32)
    src = edge_index[0].astype(jnp.int32)
    dst = edge_index[1].astype(jnp.int32)
    n = x.shape[0]
    e = src.shape[0]
    pad = E_PAD - e
    src1 = jnp.concatenate(
        [src, jnp.zeros((pad,), jnp.int32)]).reshape(16, NBLK, BLK)
    srcp = jnp.stack([src1, src1 + N_PAD])
    dstp = jnp.concatenate(
        [dst, N_PAD + (jnp.arange(pad, dtype=jnp.int32) % 8)]
    ).reshape(16, NBLK, BLK)
    xp = jnp.pad(x, ((0, N_PAD - n), (0, 0)))

    wl, bl, wr = params["Wl"], params["bl"], params["Wr"]
    n_layers = len(wl)

    cnt = _count(dstp)
    p0 = _mm0(xp, wl[0])
    q = _q0(xp, wr[0], bl[0])
    s = _segsum(p0, srcp, dstp)
    h = _combine0(s, cnt, q)
    for i in range(1, n_layers - 1):
        s = _segsum(h, srcp, dstp)
        h = _layer(s, cnt, h, wl[i], wr[i], bl[i], relu=True, out_sh=True)
    s = _segsum(h, srcp, dstp)
    out = _layer(s, cnt, h, wl[-1], wr[-1], bl[-1], relu=False, out_sh=False)
    return out[:n]


# speedup vs baseline: 1.0276x; 1.0276x over previous
import functools

import jax
import jax.numpy as jnp
from jax import lax
from jax.experimental import pallas as pl
from jax.experimental.pallas import tpu as pltpu
from jax.experimental.pallas import tpu_sc as plsc

N_PAD = 10240
HALF_STRIDE = 11264
AGG_ROWS = 10368
ROWS_PER_TILE = 648
E_PAD = 102400
EDGES_PER_TILE = 6400
BLK = 128
NBLK = EDGES_PER_TILE // BLK
R = 1024
NRB = N_PAD // R

_SC_MESH = plsc.VectorSubcoreMesh(core_axis_name="c", subcore_axis_name="s")


def _zero_tile_region(gbuf, agg, base):
    zero16 = jnp.zeros((16,), jnp.float32)

    def _zrow(i, carry):
        for c in range(8):
            gbuf[i, pl.ds(c * 16, 16)] = zero16
        return carry
    lax.fori_loop(0, BLK, _zrow, 0)
    for k in range(ROWS_PER_TILE // BLK):
        pltpu.sync_copy(gbuf.at[pl.ds(0, BLK)],
                        agg.at[pl.ds(base + k * BLK, BLK)])
    rem = ROWS_PER_TILE % BLK
    if rem:
        pltpu.sync_copy(gbuf.at[pl.ds(0, rem)],
                        agg.at[pl.ds(base + ROWS_PER_TILE - rem, rem)])


NBUF = 2


def _segsum_body(v_hbm, src_hbm, dst_hbm, out_hbm,
                 gbuf, gidx_all, didx_all, agg, gsem, ssem):
    cid = lax.axis_index("c")
    sid = lax.axis_index("s")
    base = sid * ROWS_PER_TILE

    pltpu.sync_copy(src_hbm.at[cid].at[sid], gidx_all)
    pltpu.sync_copy(dst_hbm.at[sid], didx_all)
    _zero_tile_region(gbuf, agg, base)
    plsc.subcore_barrier()

    def _slot(b):
        return gbuf.at[pl.ds(b * BLK, BLK)]

    def _gather(j, b):
        pltpu.async_copy(v_hbm.at[gidx_all.at[j]], _slot(b), gsem)

    def _gather_wait(j, b):
        pltpu.make_async_copy(v_hbm.at[gidx_all.at[j]], _slot(b), gsem).wait()

    def _scatter(j, b):
        pltpu.async_copy(_slot(b), agg.at[didx_all.at[j]], ssem, add=True)

    def _scatter_wait(j, b):
        pltpu.make_async_copy(_slot(b), agg.at[didx_all.at[j]], ssem).wait()

    for b in range(NBUF):
        _gather(b, b)

    def _round(it, carry):
        j0 = it * NBUF
        for b in range(NBUF):
            _gather_wait(j0 + b, b)
            _scatter(j0 + b, b)
        for b in range(NBUF):
            _scatter_wait(j0 + b, b)
            _gather(j0 + b + NBUF, b)
        return carry

    lax.fori_loop(0, NBLK // NBUF - 1, _round, 0)

    j0 = NBLK - NBUF
    for b in range(NBUF):
        _gather_wait(j0 + b, b)
        _scatter(j0 + b, b)
    for b in range(NBUF):
        _scatter_wait(j0 + b, b)

    plsc.subcore_barrier()

    pltpu.sync_copy(agg.at[pl.ds(base, ROWS_PER_TILE)],
                    out_hbm.at[pl.ds(cid * HALF_STRIDE + base, ROWS_PER_TILE)])


_segsum = pl.kernel(
    _segsum_body,
    mesh=_SC_MESH,
    out_type=jax.ShapeDtypeStruct((2 * HALF_STRIDE, 128), jnp.float32),
    scratch_types=[
        pltpu.VMEM((NBUF * BLK, 128), jnp.float32),
        pltpu.VMEM((NBLK, BLK), jnp.int32),
        pltpu.VMEM((NBLK, BLK), jnp.int32),
        pltpu.VMEM_SHARED((AGG_ROWS, 128), jnp.float32),
        pltpu.SemaphoreType.DMA,
        pltpu.SemaphoreType.DMA,
    ],
)


def _count_body(dst_hbm, out_hbm, gbuf, didx_all, agg, ssem):
    cid = lax.axis_index("c")
    sid = lax.axis_index("s")
    base = sid * ROWS_PER_TILE

    _zero_tile_region(gbuf, agg, base)
    pltpu.sync_copy(dst_hbm.at[sid], didx_all)

    one16 = jnp.full((16,), 1.0, jnp.float32)

    def _orow(i, carry):
        for c in range(8):
            gbuf[i, pl.ds(c * 16, 16)] = one16
        return carry
    lax.fori_loop(0, BLK, _orow, 0)
    plsc.subcore_barrier()

    ones = gbuf

    def _round(it, carry):
        j0 = it * NBUF
        for b in range(NBUF):
            pltpu.async_copy(ones, agg.at[didx_all.at[j0 + b]], ssem,
                             add=True)
        for b in range(NBUF):
            pltpu.make_async_copy(ones, agg.at[didx_all.at[j0 + b]],
                                  ssem).wait()
        return carry

    lax.fori_loop(0, NBLK // NBUF, _round, 0)
    plsc.subcore_barrier()

    pltpu.sync_copy(agg.at[pl.ds(base, ROWS_PER_TILE)],
                    out_hbm.at[pl.ds(cid * HALF_STRIDE + base, ROWS_PER_TILE)])


_count = pl.kernel(
    _count_body,
    mesh=_SC_MESH,
    out_type=jax.ShapeDtypeStruct((2 * HALF_STRIDE, 128), jnp.float32),
    scratch_types=[
        pltpu.VMEM((BLK, 128), jnp.float32),
        pltpu.VMEM((NBLK, BLK), jnp.int32),
        pltpu.VMEM_SHARED((AGG_ROWS, 128), jnp.float32),
        pltpu.SemaphoreType.DMA,
    ],
)


_DN = (((1,), (1,)), ((), ()))


def _mm0_body(x_ref, w_ref, o_ref):
    o_ref[...] = lax.dot_general(x_ref[...], w_ref[...], _DN,
                                 preferred_element_type=jnp.float32)


def _mm0(x, wl0):
    return pl.pallas_call(
        _mm0_body,
        grid=(NRB, 2),
        in_specs=[pl.BlockSpec((R, 768), lambda i, j: (i, 0)),
                  pl.BlockSpec((128, 768), lambda i, j: (j, 0))],
        out_specs=pl.BlockSpec((R, 128), lambda i, j: (j * NRB + i, 0)),
        out_shape=jax.ShapeDtypeStruct((2 * N_PAD, 128), jnp.float32),
    )(x, wl0)


def _combine0_body(s_ref, c_ref, x_ref, w_ref, b_ref, o_ref):
    rd = 1.0 / jnp.maximum(c_ref[...][:, 0:1], 1.0)
    acc = s_ref[...] * rd + b_ref[0] + lax.dot_general(
        x_ref[...], w_ref[...], _DN, preferred_element_type=jnp.float32)
    o_ref[...] = jnp.maximum(acc, 0.0)


def _combine0(s, cnt, x, w, b):
    hs_blocks = HALF_STRIDE // R
    return pl.pallas_call(
        _combine0_body,
        grid=(NRB, 2),
        in_specs=[
            pl.BlockSpec((R, 128), lambda i, j: (j * hs_blocks + i, 0)),
            pl.BlockSpec((R, 128), lambda i, j: (i, 0)),
            pl.BlockSpec((R, 768), lambda i, j: (i, 0)),
            pl.BlockSpec((128, 768), lambda i, j: (j, 0)),
            pl.BlockSpec((1, 1, 128), lambda i, j: (j, 0, 0)),
        ],
        out_specs=pl.BlockSpec((R, 128), lambda i, j: (j * NRB + i, 0)),
        out_shape=jax.ShapeDtypeStruct((2 * N_PAD, 128), jnp.float32),
    )(s, cnt, x, w, b.reshape(2, 1, 128))


def _out_spec_shape(dout, out_sh):
    nj = dout // 128
    if out_sh:
        return (pl.BlockSpec((R, 128), lambda i, j: (j * NRB + i, 0)),
                jax.ShapeDtypeStruct((2 * N_PAD, 128), jnp.float32))
    return (pl.BlockSpec((R, 128), lambda i, j: (i, j)),
            jax.ShapeDtypeStruct((N_PAD, dout), jnp.float32))


def _layer_body(relu, sA, sB, c_ref, hA, hB, wl_ref, wr_ref, b_ref, o_ref):
    rd = 1.0 / jnp.maximum(c_ref[...][:, 0:1], 1.0)
    wl = wl_ref[...]
    wr = wr_ref[...]
    acc = lax.dot_general(sA[...] * rd, wl[:, :128], _DN,
                          preferred_element_type=jnp.float32)
    acc = acc + lax.dot_general(sB[...] * rd, wl[:, 128:], _DN,
                                preferred_element_type=jnp.float32)
    acc = acc + lax.dot_general(hA[...], wr[:, :128], _DN,
                                preferred_element_type=jnp.float32)
    acc = acc + lax.dot_general(hB[...], wr[:, 128:], _DN,
                                preferred_element_type=jnp.float32)
    acc = acc + b_ref[0]
    o_ref[...] = jnp.maximum(acc, 0.0) if relu else acc


def _layer(s, cnt, h, wl, wr, b, relu, out_sh):
    dout = wl.shape[0]
    nj = dout // 128
    hs_blocks = HALF_STRIDE // R
    out_spec, out_shape = _out_spec_shape(dout, out_sh)
    return pl.pallas_call(
        functools.partial(_layer_body, relu),
        grid=(NRB, nj),
        in_specs=[
            pl.BlockSpec((R, 128), lambda i, j: (i, 0)),
            pl.BlockSpec((R, 128), lambda i, j: (hs_blocks + i, 0)),
            pl.BlockSpec((R, 128), lambda i, j: (i, 0)),
            pl.BlockSpec((R, 128), lambda i, j: (i, 0)),
            pl.BlockSpec((R, 128), lambda i, j: (NRB + i, 0)),
            pl.BlockSpec((128, 256), lambda i, j: (j, 0)),
            pl.BlockSpec((128, 256), lambda i, j: (j, 0)),
            pl.BlockSpec((1, 1, 128), lambda i, j: (j, 0, 0)),
        ],
        out_specs=out_spec,
        out_shape=out_shape,
    )(s, s, cnt, h, h, wl, wr, b.reshape(nj, 1, 128))


def kernel(x, edge_index, params):
    x = x.astype(jnp.float32)
    src = edge_index[0].astype(jnp.int32)
    dst = edge_index[1].astype(jnp.int32)
    n = x.shape[0]
    e = src.shape[0]
    pad = E_PAD - e
    src1 = jnp.concatenate(
        [src, jnp.zeros((pad,), jnp.int32)]).reshape(16, NBLK, BLK)
    srcp = jnp.stack([src1, src1 + N_PAD])
    dstp = jnp.concatenate(
        [dst, N_PAD + (jnp.arange(pad, dtype=jnp.int32) % 8)]
    ).reshape(16, NBLK, BLK)
    xp = jnp.pad(x, ((0, N_PAD - n), (0, 0)))

    wl, bl, wr = params["Wl"], params["bl"], params["Wr"]
    n_layers = len(wl)

    cnt = _count(dstp)
    p0 = _mm0(xp, wl[0])
    s = _segsum(p0, srcp, dstp)
    h = _combine0(s, cnt, xp, wr[0], bl[0])
    for i in range(1, n_layers - 1):
        s = _segsum(h, srcp, dstp)
        h = _layer(s, cnt, h, wl[i], wr[i], bl[i], relu=True, out_sh=True)
    s = _segsum(h, srcp, dstp)
    out = _layer(s, cnt, h, wl[-1], wr[-1], bl[-1], relu=False, out_sh=False)
    return out[:n]


# speedup vs baseline: 1.0623x; 1.0338x over previous
import functools

import jax
import jax.numpy as jnp
from jax import lax
from jax.experimental import pallas as pl
from jax.experimental.pallas import tpu as pltpu
from jax.experimental.pallas import tpu_sc as plsc

N_PAD = 10240
HALF_STRIDE = 12288
AGG_ROWS = 10368
ROWS_PER_TILE = 648
E_PAD = 102400
EDGES_PER_TILE = 6400
BLK = 128
NBLK = EDGES_PER_TILE // BLK
R = 2048
NRB = N_PAD // R

_SC_MESH = plsc.VectorSubcoreMesh(core_axis_name="c", subcore_axis_name="s")


def _zero_tile_region(gbuf, agg, base):
    zero16 = jnp.zeros((16,), jnp.float32)

    def _zrow(i, carry):
        for c in range(8):
            gbuf[i, pl.ds(c * 16, 16)] = zero16
        return carry
    lax.fori_loop(0, BLK, _zrow, 0)
    for k in range(ROWS_PER_TILE // BLK):
        pltpu.sync_copy(gbuf.at[pl.ds(0, BLK)],
                        agg.at[pl.ds(base + k * BLK, BLK)])
    rem = ROWS_PER_TILE % BLK
    if rem:
        pltpu.sync_copy(gbuf.at[pl.ds(0, rem)],
                        agg.at[pl.ds(base + ROWS_PER_TILE - rem, rem)])


NBUF = 2


def _segsum_body(v_hbm, src_hbm, dst_hbm, out_hbm,
                 gbuf, gidx_all, didx_all, agg, gsem, ssem):
    cid = lax.axis_index("c")
    sid = lax.axis_index("s")
    base = sid * ROWS_PER_TILE

    pltpu.sync_copy(src_hbm.at[cid].at[sid], gidx_all)
    pltpu.sync_copy(dst_hbm.at[sid], didx_all)
    _zero_tile_region(gbuf, agg, base)
    plsc.subcore_barrier()

    def _slot(b):
        return gbuf.at[pl.ds(b * BLK, BLK)]

    def _gather(j, b):
        pltpu.async_copy(v_hbm.at[gidx_all.at[j]], _slot(b), gsem)

    def _gather_wait(j, b):
        pltpu.make_async_copy(v_hbm.at[gidx_all.at[j]], _slot(b), gsem).wait()

    def _scatter(j, b):
        pltpu.async_copy(_slot(b), agg.at[didx_all.at[j]], ssem, add=True)

    def _scatter_wait(j, b):
        pltpu.make_async_copy(_slot(b), agg.at[didx_all.at[j]], ssem).wait()

    for b in range(NBUF):
        _gather(b, b)

    def _round(it, carry):
        j0 = it * NBUF
        for b in range(NBUF):
            _gather_wait(j0 + b, b)
            _scatter(j0 + b, b)
        for b in range(NBUF):
            _scatter_wait(j0 + b, b)
            _gather(j0 + b + NBUF, b)
        return carry

    lax.fori_loop(0, NBLK // NBUF - 1, _round, 0)

    j0 = NBLK - NBUF
    for b in range(NBUF):
        _gather_wait(j0 + b, b)
        _scatter(j0 + b, b)
    for b in range(NBUF):
        _scatter_wait(j0 + b, b)

    plsc.subcore_barrier()

    pltpu.sync_copy(agg.at[pl.ds(base, ROWS_PER_TILE)],
                    out_hbm.at[pl.ds(cid * HALF_STRIDE + base, ROWS_PER_TILE)])


_segsum = pl.kernel(
    _segsum_body,
    mesh=_SC_MESH,
    out_type=jax.ShapeDtypeStruct((2 * HALF_STRIDE, 128), jnp.float32),
    scratch_types=[
        pltpu.VMEM((NBUF * BLK, 128), jnp.float32),
        pltpu.VMEM((NBLK, BLK), jnp.int32),
        pltpu.VMEM((NBLK, BLK), jnp.int32),
        pltpu.VMEM_SHARED((AGG_ROWS, 128), jnp.float32),
        pltpu.SemaphoreType.DMA,
        pltpu.SemaphoreType.DMA,
    ],
)


def _count_body(dst_hbm, out_hbm, gbuf, didx_all, agg, ssem):
    cid = lax.axis_index("c")
    sid = lax.axis_index("s")
    base = sid * ROWS_PER_TILE

    _zero_tile_region(gbuf, agg, base)
    pltpu.sync_copy(dst_hbm.at[sid], didx_all)

    one16 = jnp.full((16,), 1.0, jnp.float32)

    def _orow(i, carry):
        for c in range(8):
            gbuf[i, pl.ds(c * 16, 16)] = one16
        return carry
    lax.fori_loop(0, BLK, _orow, 0)
    plsc.subcore_barrier()

    ones = gbuf

    def _round(it, carry):
        j0 = it * NBUF
        for b in range(NBUF):
            pltpu.async_copy(ones, agg.at[didx_all.at[j0 + b]], ssem,
                             add=True)
        for b in range(NBUF):
            pltpu.make_async_copy(ones, agg.at[didx_all.at[j0 + b]],
                                  ssem).wait()
        return carry

    lax.fori_loop(0, NBLK // NBUF, _round, 0)
    plsc.subcore_barrier()

    pltpu.sync_copy(agg.at[pl.ds(base, ROWS_PER_TILE)],
                    out_hbm.at[pl.ds(cid * HALF_STRIDE + base, ROWS_PER_TILE)])


_count = pl.kernel(
    _count_body,
    mesh=_SC_MESH,
    out_type=jax.ShapeDtypeStruct((2 * HALF_STRIDE, 128), jnp.float32),
    scratch_types=[
        pltpu.VMEM((BLK, 128), jnp.float32),
        pltpu.VMEM((NBLK, BLK), jnp.int32),
        pltpu.VMEM_SHARED((AGG_ROWS, 128), jnp.float32),
        pltpu.SemaphoreType.DMA,
    ],
)


_DN = (((1,), (1,)), ((), ()))


def _mm0_body(x_ref, w_ref, o_ref):
    o_ref[...] = lax.dot_general(x_ref[...], w_ref[...], _DN,
                                 preferred_element_type=jnp.float32)


def _mm0(x, wl0):
    return pl.pallas_call(
        _mm0_body,
        grid=(NRB, 2),
        in_specs=[pl.BlockSpec((R, 768), lambda i, j: (i, 0)),
                  pl.BlockSpec((128, 768), lambda i, j: (j, 0))],
        out_specs=pl.BlockSpec((R, 128), lambda i, j: (j * NRB + i, 0)),
        out_shape=jax.ShapeDtypeStruct((2 * N_PAD, 128), jnp.float32),
    )(x, wl0)


def _combine0_body(s_ref, c_ref, x_ref, w_ref, b_ref, o_ref):
    rd = 1.0 / jnp.maximum(c_ref[...][:, 0:1], 1.0)
    acc = s_ref[...] * rd + b_ref[0] + lax.dot_general(
        x_ref[...], w_ref[...], _DN, preferred_element_type=jnp.float32)
    o_ref[...] = jnp.maximum(acc, 0.0)


def _combine0(s, cnt, x, w, b):
    hs_blocks = HALF_STRIDE // R
    return pl.pallas_call(
        _combine0_body,
        grid=(NRB, 2),
        in_specs=[
            pl.BlockSpec((R, 128), lambda i, j: (j * hs_blocks + i, 0)),
            pl.BlockSpec((R, 128), lambda i, j: (i, 0)),
            pl.BlockSpec((R, 768), lambda i, j: (i, 0)),
            pl.BlockSpec((128, 768), lambda i, j: (j, 0)),
            pl.BlockSpec((1, 1, 128), lambda i, j: (j, 0, 0)),
        ],
        out_specs=pl.BlockSpec((R, 128), lambda i, j: (j * NRB + i, 0)),
        out_shape=jax.ShapeDtypeStruct((2 * N_PAD, 128), jnp.float32),
    )(s, cnt, x, w, b.reshape(2, 1, 128))


def _out_spec_shape(dout, out_sh):
    nj = dout // 128
    if out_sh:
        return (pl.BlockSpec((R, 128), lambda i, j: (j * NRB + i, 0)),
                jax.ShapeDtypeStruct((2 * N_PAD, 128), jnp.float32))
    return (pl.BlockSpec((R, 128), lambda i, j: (i, j)),
            jax.ShapeDtypeStruct((N_PAD, dout), jnp.float32))


def _layer_body(relu, sA, sB, c_ref, hA, hB, wl_ref, wr_ref, b_ref, o_ref):
    rd = 1.0 / jnp.maximum(c_ref[...][:, 0:1], 1.0)
    wl = wl_ref[...]
    wr = wr_ref[...]
    acc = lax.dot_general(sA[...] * rd, wl[:, :128], _DN,
                          preferred_element_type=jnp.float32)
    acc = acc + lax.dot_general(sB[...] * rd, wl[:, 128:], _DN,
                                preferred_element_type=jnp.float32)
    acc = acc + lax.dot_general(hA[...], wr[:, :128], _DN,
                                preferred_element_type=jnp.float32)
    acc = acc + lax.dot_general(hB[...], wr[:, 128:], _DN,
                                preferred_element_type=jnp.float32)
    acc = acc + b_ref[0]
    o_ref[...] = jnp.maximum(acc, 0.0) if relu else acc


def _layer(s, cnt, h, wl, wr, b, relu, out_sh):
    dout = wl.shape[0]
    nj = dout // 128
    hs_blocks = HALF_STRIDE // R
    out_spec, out_shape = _out_spec_shape(dout, out_sh)
    return pl.pallas_call(
        functools.partial(_layer_body, relu),
        grid=(NRB, nj),
        in_specs=[
            pl.BlockSpec((R, 128), lambda i, j: (i, 0)),
            pl.BlockSpec((R, 128), lambda i, j: (hs_blocks + i, 0)),
            pl.BlockSpec((R, 128), lambda i, j: (i, 0)),
            pl.BlockSpec((R, 128), lambda i, j: (i, 0)),
            pl.BlockSpec((R, 128), lambda i, j: (NRB + i, 0)),
            pl.BlockSpec((128, 256), lambda i, j: (j, 0)),
            pl.BlockSpec((128, 256), lambda i, j: (j, 0)),
            pl.BlockSpec((1, 1, 128), lambda i, j: (j, 0, 0)),
        ],
        out_specs=out_spec,
        out_shape=out_shape,
    )(s, s, cnt, h, h, wl, wr, b.reshape(nj, 1, 128))


def kernel(x, edge_index, params):
    x = x.astype(jnp.float32)
    src = edge_index[0].astype(jnp.int32)
    dst = edge_index[1].astype(jnp.int32)
    n = x.shape[0]
    e = src.shape[0]
    pad = E_PAD - e
    src1 = jnp.concatenate(
        [src, jnp.zeros((pad,), jnp.int32)]).reshape(16, NBLK, BLK)
    srcp = jnp.stack([src1, src1 + N_PAD])
    dstp = jnp.concatenate(
        [dst, N_PAD + (jnp.arange(pad, dtype=jnp.int32) % 8)]
    ).reshape(16, NBLK, BLK)
    xp = jnp.pad(x, ((0, N_PAD - n), (0, 0)))

    wl, bl, wr = params["Wl"], params["bl"], params["Wr"]
    n_layers = len(wl)

    cnt = _count(dstp)
    p0 = _mm0(xp, wl[0])
    s = _segsum(p0, srcp, dstp)
    h = _combine0(s, cnt, xp, wr[0], bl[0])
    for i in range(1, n_layers - 1):
        s = _segsum(h, srcp, dstp)
        h = _layer(s, cnt, h, wl[i], wr[i], bl[i], relu=True, out_sh=True)
    s = _segsum(h, srcp, dstp)
    out = _layer(s, cnt, h, wl[-1], wr[-1], bl[-1], relu=False, out_sh=False)
    return out[:n]
